# baseline (speedup 1.0000x reference)
import functools

import jax
import jax.numpy as jnp
from jax import lax
from jax.experimental import pallas as pl
from jax.experimental.pallas import tpu as pltpu
from jax.experimental.pallas import tpu_sc as plsc

B = 1024
K = 1000
D = 128
OUT = 64
TOP_K = 10
KP = 1024
BB = 256
NW = 32
RPW = B // NW
NCH = KP // 16
ROWG = 4

_NEG = -3.0e38


def _lp_body(x_ref, mean_ref, stddev_ref, lp_ref):
    x = x_ref[...]
    mean = mean_ref[...]
    std = stddev_ref[...]

    iv = 1.0 / (std * std)
    w2 = mean * iv
    bias = (jnp.sum(-0.5 * mean * w2 - jnp.log(std), axis=1)
            - 0.5 * D * jnp.log(2.0 * jnp.pi))

    t1 = jax.lax.dot_general(x * x, iv, (((1,), (1,)), ((), ())),
                             preferred_element_type=jnp.float32,
                             precision=jax.lax.Precision.HIGHEST)
    t2 = jax.lax.dot_general(x, w2, (((1,), (1,)), ((), ())),
                             preferred_element_type=jnp.float32,
                             precision=jax.lax.Precision.HIGHEST)
    lp = -0.5 * t1 + t2 + bias[None, :]
    lp_ref[...] = jnp.concatenate(
        [lp, jnp.full((BB, KP - K), _NEG, jnp.float32)], axis=1)


def _distances(x, mean, stddev):
    return pl.pallas_call(
        _lp_body,
        grid=(B // BB,),
        in_specs=[
            pl.BlockSpec((BB, D), lambda i: (i, 0)),
            pl.BlockSpec((K, D), lambda i: (0, 0)),
            pl.BlockSpec((K, D), lambda i: (0, 0)),
        ],
        out_specs=pl.BlockSpec((BB, KP), lambda i: (i, 0)),
        out_shape=jax.ShapeDtypeStruct((B, KP), jnp.float32),
        compiler_params=pltpu.CompilerParams(
            dimension_semantics=("arbitrary",)),
    )(x, mean, stddev)


def _lane_bcast(v, j):
    idx = jnp.full((16, 1), j, jnp.int32)
    dn = lax.GatherDimensionNumbers(
        offset_dims=(), collapsed_slice_dims=(0,), start_index_map=(0,))
    return lax.gather(v, idx, dn, (1,),
                      mode=lax.GatherScatterMode.PROMISE_IN_BOUNDS)


def _make_sc_combine():
    mesh = plsc.VectorSubcoreMesh(core_axis_name="c", subcore_axis_name="s",
                                  num_cores=2, num_subcores=16)

    @functools.partial(
        pl.kernel,
        out_type=jax.ShapeDtypeStruct((B, OUT), jnp.float32),
        mesh=mesh,
        scratch_types=[
            pltpu.VMEM((RPW, KP), jnp.float32),
            pltpu.VMEM((RPW * 16,), jnp.int32),
            pltpu.VMEM((RPW, 16), jnp.float32),
            pltpu.VMEM((RPW * 16, 128), jnp.float32),
            pltpu.VMEM((RPW, OUT), jnp.float32),
            pltpu.SemaphoreType.DMA,
        ],
        compiler_params=pltpu.CompilerParams(needs_layout_passes=False),
    )
    def sc_top10(lp_hbm, outputs_hbm, out_hbm,
                 rows_v, idx_v, w_v, gath_v, acc_v, sem):
        wid = lax.axis_index("s") * 2 + lax.axis_index("c")
        base = pl.multiple_of(wid * RPW, RPW)
        pltpu.sync_copy(lp_hbm.at[pl.ds(base, RPW)], rows_v)

        lane = lax.broadcasted_iota(jnp.int32, (16,), 0)

        for g in range(RPW // ROWG):
            rows = [g * ROWG + r for r in range(ROWG)]
            carry = []
            for r in rows:
                v0 = rows_v[r, pl.ds(0, 16)]
                tv, ti = plsc.sort_key_val(v0, lane)
                carry += [tv, ti]

            def body(c, cr, rows=rows):
                cbase = c * 16
                idx = lane + cbase
                out = []
                for q, r in enumerate(rows):
                    tv, ti = cr[2 * q], cr[2 * q + 1]
                    v = rows_v[r, pl.ds(pl.multiple_of(cbase, 16), 16)]
                    sv, si = plsc.sort_key_val(v, idx, descending=True)
                    m = tv >= sv
                    nv = jnp.where(m, tv, sv)
                    ni = jnp.where(m, ti, si)
                    tv, ti = plsc.sort_key_val(nv, ni)
                    out += [tv, ti]
                return tuple(out)

            carry = lax.fori_loop(1, NCH, body, tuple(carry))

            for q, r in enumerate(rows):
                tv, ti = carry[2 * q], carry[2 * q + 1]
                m0 = jnp.max(tv)
                e = jnp.where(lane >= 16 - TOP_K,
                              jnp.exp(tv - m0), 0.0)
                w_v[r] = e / jnp.sum(e)
                idx_v[pl.ds(r * 16, 16)] = ti

        pltpu.async_copy(outputs_hbm.at[idx_v], gath_v, sem).wait()

        def crow(r, _):
            wvec = w_v[r]
            for c in range(OUT // 16):
                acc = jnp.zeros((16,), jnp.float32)
                for j in range(16 - TOP_K, 16):
                    wj = _lane_bcast(wvec, j)
                    acc = acc + wj * gath_v[r * 16 + j, pl.ds(c * 16, 16)]
                acc_v[r, pl.ds(c * 16, 16)] = acc
            return 0

        lax.fori_loop(0, RPW, crow, 0)
        pltpu.sync_copy(acc_v, out_hbm.at[pl.ds(base, RPW)])

    return sc_top10


_sc_combine = _make_sc_combine()


@jax.jit
def kernel(x, mean, stddev, outputs):
    lp = _distances(x, mean, stddev)
    outputs_p = jnp.concatenate(
        [outputs, jnp.zeros((K, 128 - OUT), jnp.float32)], axis=1)
    return _sc_combine(lp, outputs_p)
